# baseline (speedup 1.0000x reference)
import functools

import jax
import jax.numpy as jnp
from jax import lax
from jax.experimental import pallas as pl
from jax.experimental.pallas import tpu as pltpu
from jax.experimental.pallas import tpu_sc as plsc

EMBED_DIM = 64
BATCH = 4096
SEQ_LEN = 200

_info = plsc.get_sparse_core_info()
NC, NS, L = _info.num_cores, _info.num_subcores, _info.num_lanes
NW = NC * NS

ROWS_PER_W = BATCH // NW
TOK_PER_W = ROWS_PER_W * SEQ_LEN
GCHUNK_A = 128
GCHUNK_B = SEQ_LEN - GCHUNK_A
NBUF = 4
NITER = ROWS_PER_W // NBUF
NFULL = SEQ_LEN // L
TAIL_OFF = SEQ_LEN - L


def _sc_body(inp_hbm, w_hbm, out_hbm, inp_v, idx_v, rows_v, gsem, ssem):
    wid = lax.axis_index("s") * NC + lax.axis_index("c")
    row0 = wid * ROWS_PER_W

    pltpu.sync_copy(inp_hbm.at[pl.ds(row0, ROWS_PER_W), :], inp_v)

    def row_body(r, _):
        base = pl.multiple_of(r * SEQ_LEN, 8)
        carry = jnp.int32(0)
        for k in range(NFULL):
            x = inp_v[r, pl.ds(k * L, L)]
            m = jnp.minimum(jnp.abs(x), 1)
            c = plsc.cumsum(m)
            idx_v[pl.ds(base + k * L, L)] = (c + carry) * m
            carry = carry + jnp.max(c)
        x = inp_v[r, pl.ds(TAIL_OFF, L)]
        m = jnp.minimum(jnp.abs(x), 1)
        c = plsc.cumsum(m)
        lane = lax.iota(jnp.int32, L)
        nlap = L - (SEQ_LEN - NFULL * L)
        c7 = jnp.max(jnp.where(lane < nlap, c, 0))
        idx_v[pl.ds(base + TAIL_OFF, L)] = (carry - c7 + c) * m
        return 0

    lax.fori_loop(0, ROWS_PER_W, row_body, 0)

    def gather_a(b, r):
        off = pl.multiple_of(r * SEQ_LEN, 8)
        return pltpu.make_async_copy(
            w_hbm.at[idx_v.at[pl.ds(off, GCHUNK_A)]],
            rows_v.at[b, pl.ds(0, GCHUNK_A), :],
            gsem.at[b],
        )

    def gather_b(b, r):
        off = pl.multiple_of(r * SEQ_LEN + GCHUNK_A, 8)
        return pltpu.make_async_copy(
            w_hbm.at[idx_v.at[pl.ds(off, GCHUNK_B)]],
            rows_v.at[b, pl.ds(GCHUNK_A, GCHUNK_B), :],
            gsem.at[b],
        )

    def scatter_d(b, r):
        return pltpu.make_async_copy(
            rows_v.at[b], out_hbm.at[row0 + r], ssem.at[b]
        )

    for b in range(NBUF):
        gather_a(b, b).start()
        gather_b(b, b).start()

    @pl.loop(0, NITER)
    def ring(i):
        r0 = i * NBUF
        for b in range(NBUF):
            r = r0 + b
            gather_a(b, r).wait()
            gather_b(b, r).wait()
            scatter_d(b, r).start()
        for b in range(NBUF):
            r = r0 + b
            scatter_d(b, r).wait()

            @pl.when(i < NITER - 1)
            def _():
                gather_a(b, r + NBUF).start()
                gather_b(b, r + NBUF).start()


@functools.partial(
    pl.kernel,
    out_type=jax.ShapeDtypeStruct((BATCH, SEQ_LEN, EMBED_DIM), jnp.float32),
    mesh=plsc.VectorSubcoreMesh(core_axis_name="c", subcore_axis_name="s"),
    scratch_types=[
        pltpu.VMEM((ROWS_PER_W, SEQ_LEN), jnp.int32),
        pltpu.VMEM((TOK_PER_W,), jnp.int32),
        pltpu.VMEM((NBUF, SEQ_LEN, EMBED_DIM), jnp.float32),
        pltpu.SemaphoreType.DMA((NBUF,)),
        pltpu.SemaphoreType.DMA((NBUF,)),
    ],
    compiler_params=pltpu.CompilerParams(
        use_tc_tiling_on_sc=False, needs_layout_passes=False
    ),
)
def _embed_lookup(inp_hbm, w_hbm, out_hbm, inp_v, idx_v, rows_v, gsem, ssem):
    _sc_body(inp_hbm, w_hbm, out_hbm, inp_v, idx_v, rows_v, gsem, ssem)


def kernel(input, weights):
    inp = input.astype(jnp.int32)
    return _embed_lookup(inp, weights.astype(jnp.float32))


# speedup vs baseline: 1.5312x; 1.5312x over previous
import functools

import jax
import jax.numpy as jnp
from jax import lax
from jax.experimental import pallas as pl
from jax.experimental.pallas import tpu as pltpu
from jax.experimental.pallas import tpu_sc as plsc

EMBED_DIM = 64
BATCH = 4096
SEQ_LEN = 200

_info = plsc.get_sparse_core_info()
NC, NS, L = _info.num_cores, _info.num_subcores, _info.num_lanes
NW = NC * NS

ROWS_PER_W = BATCH // NW
TOK_PER_W = ROWS_PER_W * SEQ_LEN
GCHUNK_A = 128
GCHUNK_B = SEQ_LEN - GCHUNK_A
NBUF = 4
NITER = ROWS_PER_W // NBUF
NFULL = SEQ_LEN // L
TAIL_OFF = SEQ_LEN - L


TBL_ROWS = 256


def _sc_body(inp_hbm, w_hbm, out_hbm, rep_hbm, inp_v, idx_v, rows_v, gsem, ssem):
    wid = lax.axis_index("s") * NC + lax.axis_index("c")
    row0 = wid * ROWS_PER_W
    tbl0 = wid * TBL_ROWS

    for h in range(2):
        half = TBL_ROWS // 2
        pltpu.sync_copy(
            w_hbm.at[pl.ds(h * half, half), :],
            rows_v.at[h, pl.ds(0, half), :],
        )
        pltpu.sync_copy(
            rows_v.at[h, pl.ds(0, half), :],
            rep_hbm.at[pl.ds(tbl0 + h * half, half), :],
        )

    pltpu.sync_copy(inp_hbm.at[pl.ds(row0, ROWS_PER_W), :], inp_v)

    def row_body(r, _):
        base = pl.multiple_of(r * SEQ_LEN, 8)
        carry = jnp.int32(0)
        for k in range(NFULL):
            x = inp_v[r, pl.ds(k * L, L)]
            m = jnp.minimum(jnp.abs(x), 1)
            c = plsc.cumsum(m)
            idx_v[pl.ds(base + k * L, L)] = (c + carry) * m + tbl0
            carry = carry + jnp.max(c)
        x = inp_v[r, pl.ds(TAIL_OFF, L)]
        m = jnp.minimum(jnp.abs(x), 1)
        c = plsc.cumsum(m)
        lane = lax.iota(jnp.int32, L)
        nlap = L - (SEQ_LEN - NFULL * L)
        c7 = jnp.max(jnp.where(lane < nlap, c, 0))
        idx_v[pl.ds(base + TAIL_OFF, L)] = (carry - c7 + c) * m + tbl0
        return 0

    lax.fori_loop(0, ROWS_PER_W, row_body, 0)

    def gather_a(b, r):
        off = pl.multiple_of(r * SEQ_LEN, 8)
        return pltpu.make_async_copy(
            rep_hbm.at[idx_v.at[pl.ds(off, GCHUNK_A)]],
            rows_v.at[b, pl.ds(0, GCHUNK_A), :],
            gsem.at[b],
        )

    def gather_b(b, r):
        off = pl.multiple_of(r * SEQ_LEN + GCHUNK_A, 8)
        return pltpu.make_async_copy(
            rep_hbm.at[idx_v.at[pl.ds(off, GCHUNK_B)]],
            rows_v.at[b, pl.ds(GCHUNK_A, GCHUNK_B), :],
            gsem.at[b],
        )

    def scatter_d(b, r):
        return pltpu.make_async_copy(
            rows_v.at[b], out_hbm.at[row0 + r], ssem.at[b]
        )

    for b in range(NBUF):
        gather_a(b, b).start()
        gather_b(b, b).start()

    @pl.loop(0, NITER)
    def ring(i):
        r0 = i * NBUF
        for b in range(NBUF):
            r = r0 + b
            gather_a(b, r).wait()
            gather_b(b, r).wait()
            scatter_d(b, r).start()
        for b in range(NBUF):
            r = r0 + b
            scatter_d(b, r).wait()

            @pl.when(i < NITER - 1)
            def _():
                gather_a(b, r + NBUF).start()
                gather_b(b, r + NBUF).start()


@functools.partial(
    pl.kernel,
    out_type=(
        jax.ShapeDtypeStruct((BATCH, SEQ_LEN, EMBED_DIM), jnp.float32),
        jax.ShapeDtypeStruct((NW * TBL_ROWS, EMBED_DIM), jnp.float32),
    ),
    mesh=plsc.VectorSubcoreMesh(core_axis_name="c", subcore_axis_name="s"),
    scratch_types=[
        pltpu.VMEM((ROWS_PER_W, SEQ_LEN), jnp.int32),
        pltpu.VMEM((TOK_PER_W,), jnp.int32),
        pltpu.VMEM((NBUF, SEQ_LEN, EMBED_DIM), jnp.float32),
        pltpu.SemaphoreType.DMA((NBUF,)),
        pltpu.SemaphoreType.DMA((NBUF,)),
    ],
    compiler_params=pltpu.CompilerParams(
        use_tc_tiling_on_sc=False, needs_layout_passes=False
    ),
)
def _embed_lookup(inp_hbm, w_hbm, out_hbm, rep_hbm, inp_v, idx_v, rows_v, gsem, ssem):
    _sc_body(inp_hbm, w_hbm, out_hbm, rep_hbm, inp_v, idx_v, rows_v, gsem, ssem)


def kernel(input, weights):
    inp = input.astype(jnp.int32)
    out, _ = _embed_lookup(inp, weights.astype(jnp.float32))
    return out


# speedup vs baseline: 1.5533x; 1.0144x over previous
import functools

import jax
import jax.numpy as jnp
from jax import lax
from jax.experimental import pallas as pl
from jax.experimental.pallas import tpu as pltpu
from jax.experimental.pallas import tpu_sc as plsc

EMBED_DIM = 64
BATCH = 4096
SEQ_LEN = 200

_info = plsc.get_sparse_core_info()
NC, NS, L = _info.num_cores, _info.num_subcores, _info.num_lanes
NW = NC * NS

ROWS_PER_W = BATCH // NW
TOK_PER_W = ROWS_PER_W * SEQ_LEN
GCHUNK_A = 128
GCHUNK_B = SEQ_LEN - GCHUNK_A
NBUF = 4
NITER = ROWS_PER_W // NBUF
NFULL = SEQ_LEN // L
TAIL_OFF = SEQ_LEN - L
TBL_ROWS = 256


def _sc_body(inp_hbm, rep_hbm, out_hbm, inp_v, idx_v, rows_v, gsem, ssem):
    wid = lax.axis_index("s") * NC + lax.axis_index("c")
    row0 = wid * ROWS_PER_W
    tok0 = pl.multiple_of(row0 * SEQ_LEN, 8)
    tbl0 = wid * TBL_ROWS

    pltpu.sync_copy(inp_hbm.at[pl.ds(tok0, TOK_PER_W)], inp_v)

    def row_compute(r):
        base = pl.multiple_of(r * SEQ_LEN, 8)
        carry = jnp.int32(0)
        for k in range(NFULL):
            x = inp_v[pl.ds(base + k * L, L)]
            m = jnp.minimum(jnp.abs(x), 1)
            c = plsc.cumsum(m)
            idx_v[pl.ds(base + k * L, L)] = (c + carry) * m + tbl0
            carry = carry + jnp.max(c)
        x = inp_v[pl.ds(base + TAIL_OFF, L)]
        m = jnp.minimum(jnp.abs(x), 1)
        c = plsc.cumsum(m)
        lane = lax.iota(jnp.int32, L)
        nlap = L - (SEQ_LEN - NFULL * L)
        c7 = jnp.max(jnp.where(lane < nlap, c, 0))
        idx_v[pl.ds(base + TAIL_OFF, L)] = (carry - c7 + c) * m + tbl0

    def gather_a(b, r):
        off = pl.multiple_of(r * SEQ_LEN, 8)
        return pltpu.make_async_copy(
            rep_hbm.at[idx_v.at[pl.ds(off, GCHUNK_A)]],
            rows_v.at[b, pl.ds(0, GCHUNK_A), :],
            gsem.at[b],
        )

    def gather_b(b, r):
        off = pl.multiple_of(r * SEQ_LEN + GCHUNK_A, 8)
        return pltpu.make_async_copy(
            rep_hbm.at[idx_v.at[pl.ds(off, GCHUNK_B)]],
            rows_v.at[b, pl.ds(GCHUNK_A, GCHUNK_B), :],
            gsem.at[b],
        )

    def scatter_d(b, r):
        return pltpu.make_async_copy(
            rows_v.at[b], out_hbm.at[row0 + r], ssem.at[b]
        )

    for b in range(NBUF):
        row_compute(b)
    for b in range(NBUF):
        gather_a(b, b).start()
        gather_b(b, b).start()

    @pl.loop(0, NITER)
    def ring(i):
        r0 = i * NBUF
        @pl.when(i < NITER - 1)
        def _():
            for b in range(NBUF):
                row_compute(r0 + NBUF + b)

        for b in range(NBUF):
            r = r0 + b
            gather_a(b, r).wait()
            gather_b(b, r).wait()
            scatter_d(b, r).start()
        for b in range(NBUF):
            r = r0 + b
            scatter_d(b, r).wait()

            @pl.when(i < NITER - 1)
            def _():
                gather_a(b, r + NBUF).start()
                gather_b(b, r + NBUF).start()


@functools.partial(
    pl.kernel,
    out_type=jax.ShapeDtypeStruct((BATCH, SEQ_LEN, EMBED_DIM), jnp.float32),
    mesh=plsc.VectorSubcoreMesh(core_axis_name="c", subcore_axis_name="s"),
    scratch_types=[
        pltpu.VMEM((TOK_PER_W,), jnp.int32),
        pltpu.VMEM((TOK_PER_W,), jnp.int32),
        pltpu.VMEM((NBUF, SEQ_LEN, EMBED_DIM), jnp.float32),
        pltpu.SemaphoreType.DMA((NBUF,)),
        pltpu.SemaphoreType.DMA((NBUF,)),
    ],
    compiler_params=pltpu.CompilerParams(
        use_tc_tiling_on_sc=False, needs_layout_passes=False
    ),
)
def _embed_lookup(inp_hbm, rep_hbm, out_hbm, inp_v, idx_v, rows_v, gsem, ssem):
    _sc_body(inp_hbm, rep_hbm, out_hbm, inp_v, idx_v, rows_v, gsem, ssem)


def kernel(input, weights):
    inp = input.astype(jnp.int32).reshape(BATCH * SEQ_LEN)
    rep = jnp.tile(weights[:TBL_ROWS].astype(jnp.float32), (NW, 1))
    return _embed_lookup(inp, rep)


# speedup vs baseline: 2.7826x; 1.7914x over previous
import functools

import jax
import jax.numpy as jnp
from jax import lax
from jax.experimental import pallas as pl
from jax.experimental.pallas import tpu as pltpu
from jax.experimental.pallas import tpu_sc as plsc

EMBED_DIM = 64
BATCH = 4096
SEQ_LEN = 200

_info = plsc.get_sparse_core_info()
NC, NS, L = _info.num_cores, _info.num_subcores, _info.num_lanes
NW = NC * NS

ROWS_PER_W = BATCH // NW
TOK_PER_W = ROWS_PER_W * SEQ_LEN
GCHUNK_A = 128
GCHUNK_B = SEQ_LEN - GCHUNK_A
NBUF = 4
NITER = ROWS_PER_W // NBUF
NFULL = SEQ_LEN // L
TAIL_OFF = SEQ_LEN - L
TBL_ROWS = 256


def _sc_body(inp_hbm, rep_hbm, out_hbm, inp_v, idx_v, rows_v, gsem, ssem):
    wid = lax.axis_index("s") * NC + lax.axis_index("c")
    row0 = wid * ROWS_PER_W
    tok0 = pl.multiple_of(row0 * SEQ_LEN, 8)
    tbl0 = wid * TBL_ROWS

    pltpu.sync_copy(inp_hbm.at[pl.ds(tok0, TOK_PER_W)], inp_v)

    def row_compute(r):
        base = pl.multiple_of(r * SEQ_LEN, 8)
        carry = jnp.int32(0)
        for k in range(NFULL):
            x = inp_v[pl.ds(base + k * L, L)]
            m = jnp.minimum(jnp.abs(x), 1)
            c = plsc.cumsum(m)
            idx_v[pl.ds(base + k * L, L)] = (c + carry) * m + tbl0
            carry = carry + jnp.max(c)
        x = inp_v[pl.ds(base + TAIL_OFF, L)]
        m = jnp.minimum(jnp.abs(x), 1)
        c = plsc.cumsum(m)
        lane = lax.iota(jnp.int32, L)
        nlap = L - (SEQ_LEN - NFULL * L)
        c7 = jnp.max(jnp.where(lane < nlap, c, 0))
        idx_v[pl.ds(base + TAIL_OFF, L)] = (carry - c7 + c) * m + tbl0

    def gather_a(b, r):
        off = pl.multiple_of(r * SEQ_LEN, 8)
        return pltpu.make_async_copy(
            rep_hbm.at[idx_v.at[pl.ds(off, GCHUNK_A)]],
            rows_v.at[b, pl.ds(0, GCHUNK_A), :],
            gsem.at[b],
        )

    def gather_b(b, r):
        off = pl.multiple_of(r * SEQ_LEN + GCHUNK_A, 8)
        return pltpu.make_async_copy(
            rep_hbm.at[idx_v.at[pl.ds(off, GCHUNK_B)]],
            rows_v.at[b, pl.ds(GCHUNK_A, GCHUNK_B), :],
            gsem.at[b],
        )

    def scatter_d(b, r):
        return pltpu.make_async_copy(
            rows_v.at[b],
            out_hbm.at[row0 + r, :, pl.ds(0, EMBED_DIM)],
            ssem.at[b],
        )

    for b in range(NBUF):
        row_compute(b)
    for b in range(NBUF):
        gather_a(b, b).start()
        gather_b(b, b).start()

    @pl.loop(0, NITER)
    def ring(i):
        r0 = i * NBUF
        @pl.when(i < NITER - 1)
        def _():
            for b in range(NBUF):
                row_compute(r0 + NBUF + b)

        for b in range(NBUF):
            r = r0 + b
            gather_a(b, r).wait()
            gather_b(b, r).wait()
            scatter_d(b, r).start()
        for b in range(NBUF):
            r = r0 + b
            scatter_d(b, r).wait()

            @pl.when(i < NITER - 1)
            def _():
                gather_a(b, r + NBUF).start()
                gather_b(b, r + NBUF).start()


@functools.partial(
    pl.kernel,
    out_type=jax.ShapeDtypeStruct((BATCH, SEQ_LEN, 128), jnp.float32),
    mesh=plsc.VectorSubcoreMesh(core_axis_name="c", subcore_axis_name="s"),
    scratch_types=[
        pltpu.VMEM((TOK_PER_W,), jnp.int32),
        pltpu.VMEM((TOK_PER_W,), jnp.int32),
        pltpu.VMEM((NBUF, SEQ_LEN, EMBED_DIM), jnp.float32),
        pltpu.SemaphoreType.DMA((NBUF,)),
        pltpu.SemaphoreType.DMA((NBUF,)),
    ],
    compiler_params=pltpu.CompilerParams(
        use_tc_tiling_on_sc=False, needs_layout_passes=False
    ),
)
def _embed_lookup(inp_hbm, rep_hbm, out_hbm, inp_v, idx_v, rows_v, gsem, ssem):
    _sc_body(inp_hbm, rep_hbm, out_hbm, inp_v, idx_v, rows_v, gsem, ssem)


def kernel(input, weights):
    inp = input.astype(jnp.int32).reshape(BATCH * SEQ_LEN)
    rep = jnp.tile(weights[:TBL_ROWS].astype(jnp.float32), (NW, 1))
    return _embed_lookup(inp, rep)[:, :, :EMBED_DIM]


# speedup vs baseline: 2.7862x; 1.0013x over previous
import functools

import jax
import jax.numpy as jnp
from jax import lax
from jax.experimental import pallas as pl
from jax.experimental.pallas import tpu as pltpu
from jax.experimental.pallas import tpu_sc as plsc

EMBED_DIM = 64
BATCH = 4096
SEQ_LEN = 200

_info = plsc.get_sparse_core_info()
NC, NS, L = _info.num_cores, _info.num_subcores, _info.num_lanes
NW = NC * NS

ROWS_PER_W = BATCH // NW
TOK_PER_W = ROWS_PER_W * SEQ_LEN
GCHUNK_A = 128
GCHUNK_B = SEQ_LEN - GCHUNK_A
NBUF = 4
NITER = ROWS_PER_W // NBUF
NFULL = SEQ_LEN // L
TAIL_OFF = SEQ_LEN - L
TBL_ROWS = 256


def _sc_body(inp_hbm, rep_hbm, out_hbm, inp_v, idx_v, rows_v, gsem, ssem):
    wid = lax.axis_index("s") * NC + lax.axis_index("c")
    row0 = wid * ROWS_PER_W
    tok0 = pl.multiple_of(row0 * SEQ_LEN, 8)
    tbl0 = wid * TBL_ROWS

    pltpu.sync_copy(inp_hbm.at[pl.ds(tok0, TOK_PER_W)], inp_v)

    def row_compute(r):
        base = pl.multiple_of(r * SEQ_LEN, 8)
        carry = jnp.int32(0)
        for k in range(NFULL):
            x = inp_v[pl.ds(base + k * L, L)]
            m = jnp.minimum(jnp.abs(x), 1)
            c = plsc.cumsum(m)
            idx_v[pl.ds(base + k * L, L)] = (c + carry) * m + tbl0
            carry = carry + jnp.max(c)
        x = inp_v[pl.ds(base + TAIL_OFF, L)]
        m = jnp.minimum(jnp.abs(x), 1)
        c = plsc.cumsum(m)
        lane = lax.iota(jnp.int32, L)
        nlap = L - (SEQ_LEN - NFULL * L)
        c7 = jnp.max(jnp.where(lane < nlap, c, 0))
        idx_v[pl.ds(base + TAIL_OFF, L)] = (carry - c7 + c) * m + tbl0

    def gather_a(b, r):
        off = pl.multiple_of(r * SEQ_LEN, 8)
        return pltpu.make_async_copy(
            rep_hbm.at[idx_v.at[pl.ds(off, GCHUNK_A)]],
            rows_v.at[b, pl.ds(0, GCHUNK_A), :],
            gsem.at[b],
        )

    def gather_b(b, r):
        off = pl.multiple_of(r * SEQ_LEN + GCHUNK_A, 8)
        return pltpu.make_async_copy(
            rep_hbm.at[idx_v.at[pl.ds(off, GCHUNK_B)]],
            rows_v.at[b, pl.ds(GCHUNK_A, GCHUNK_B), :],
            gsem.at[b],
        )

    def scatter_d(b, r):
        return pltpu.make_async_copy(
            rows_v.at[b],
            out_hbm.at[row0 + r, :, pl.ds(0, EMBED_DIM)],
            ssem.at[b],
        )

    for b in range(NBUF):
        row_compute(b)
    for b in range(NBUF):
        gather_a(b, b).start()
        gather_b(b, b).start()

    @pl.loop(0, NITER)
    def ring(i):
        r0 = i * NBUF
        @pl.when(i < NITER - 1)
        def _():
            for b in range(NBUF):
                row_compute(r0 + NBUF + b)

        for b in range(NBUF):
            r = r0 + b
            gather_a(b, r).wait()
            gather_b(b, r).wait()
            scatter_d(b, r).start()
        for b in range(NBUF):
            r = r0 + b
            scatter_d(b, r).wait()

            @pl.when(i < NITER - 1)
            def _():
                gather_a(b, r + NBUF).start()
                gather_b(b, r + NBUF).start()


@functools.partial(
    pl.kernel,
    out_type=jax.ShapeDtypeStruct((BATCH, SEQ_LEN, 128), jnp.float32),
    mesh=plsc.VectorSubcoreMesh(core_axis_name="c", subcore_axis_name="s"),
    scratch_types=[
        pltpu.VMEM((TOK_PER_W,), jnp.int32),
        pltpu.VMEM((TOK_PER_W,), jnp.int32),
        pltpu.VMEM((NBUF, SEQ_LEN, EMBED_DIM), jnp.float32),
        pltpu.SemaphoreType.DMA((NBUF,)),
        pltpu.SemaphoreType.DMA((NBUF,)),
    ],
    compiler_params=pltpu.CompilerParams(
        use_tc_tiling_on_sc=False, needs_layout_passes=False
    ),
)
def _embed_lookup(inp_hbm, rep_hbm, out_hbm, inp_v, idx_v, rows_v, gsem, ssem):
    _sc_body(inp_hbm, rep_hbm, out_hbm, inp_v, idx_v, rows_v, gsem, ssem)


def kernel(input, weights):
    inp = input.astype(jnp.int32).reshape(BATCH * SEQ_LEN)
    rep = jnp.tile(weights[:TBL_ROWS].astype(jnp.float32), (NW, 1))
    return _embed_lookup(inp, rep)[:, :, :EMBED_DIM] * jnp.float32(1.0)


# speedup vs baseline: 2.8198x; 1.0121x over previous
import functools

import jax
import jax.numpy as jnp
from jax import lax
from jax.experimental import pallas as pl
from jax.experimental.pallas import tpu as pltpu
from jax.experimental.pallas import tpu_sc as plsc

EMBED_DIM = 64
BATCH = 4096
SEQ_LEN = 200

_info = plsc.get_sparse_core_info()
NC, NS, L = _info.num_cores, _info.num_subcores, _info.num_lanes
NW = NC * NS

ROWS_PER_W = BATCH // NW
TOK_PER_W = ROWS_PER_W * SEQ_LEN
GCHUNK_A = 128
GCHUNK_B = SEQ_LEN - GCHUNK_A
NBUF = 6
NSTEP = ROWS_PER_W // 2
NFULL = SEQ_LEN // L
TAIL_OFF = SEQ_LEN - L
TBL_ROWS = 256


def _sc_body(inp_hbm, rep_hbm, out_hbm, inp_v, idx_v, rows_v, gsem, ssem):
    wid = lax.axis_index("s") * NC + lax.axis_index("c")
    row0 = wid * ROWS_PER_W
    tok0 = pl.multiple_of(row0 * SEQ_LEN, 8)
    tbl0 = wid * TBL_ROWS

    pltpu.sync_copy(inp_hbm.at[pl.ds(tok0, TOK_PER_W)], inp_v)

    def row_compute(r):
        base = pl.multiple_of(r * SEQ_LEN, 8)
        carry = jnp.int32(0)
        for k in range(NFULL):
            x = inp_v[pl.ds(base + k * L, L)]
            m = jnp.minimum(jnp.abs(x), 1)
            c = plsc.cumsum(m)
            idx_v[pl.ds(base + k * L, L)] = (c + carry) * m + tbl0
            carry = carry + jnp.max(c)
        x = inp_v[pl.ds(base + TAIL_OFF, L)]
        m = jnp.minimum(jnp.abs(x), 1)
        c = plsc.cumsum(m)
        lane = lax.iota(jnp.int32, L)
        nlap = L - (SEQ_LEN - NFULL * L)
        c7 = jnp.max(jnp.where(lane < nlap, c, 0))
        idx_v[pl.ds(base + TAIL_OFF, L)] = (carry - c7 + c) * m + tbl0

    def gather_a(b, r):
        off = pl.multiple_of(r * SEQ_LEN, 8)
        return pltpu.make_async_copy(
            rep_hbm.at[idx_v.at[pl.ds(off, GCHUNK_A)]],
            rows_v.at[b, pl.ds(0, GCHUNK_A), :],
            gsem.at[b],
        )

    def gather_b(b, r):
        off = pl.multiple_of(r * SEQ_LEN + GCHUNK_A, 8)
        return pltpu.make_async_copy(
            rep_hbm.at[idx_v.at[pl.ds(off, GCHUNK_B)]],
            rows_v.at[b, pl.ds(GCHUNK_A, GCHUNK_B), :],
            gsem.at[b],
        )

    def scatter_d(b, r):
        return pltpu.make_async_copy(
            rows_v.at[b],
            out_hbm.at[row0 + r, :, pl.ds(0, EMBED_DIM)],
            ssem.at[b],
        )

    def start_pair_gathers(pair, r):
        for m in range(2):
            gather_a(2 * pair + m, r + m).start()
            gather_b(2 * pair + m, r + m).start()

    def step(j, u):
        r = 2 * j

        @pl.when(j < NSTEP - 2)
        def _():
            row_compute(r + 4)
            row_compute(r + 5)

        for m in range(2):
            b = 2 * u + m
            gather_a(b, r + m).wait()
            gather_b(b, r + m).wait()
            scatter_d(b, r + m).start()

        prev = (u - 1) % 3

        @pl.when(j > 0)
        def _():
            for m in range(2):
                scatter_d(2 * prev + m, r - 2 + m).wait()

        @pl.when(j < NSTEP - 2)
        def _():
            start_pair_gathers((u + 2) % 3, r + 4)

    for r in range(4):
        row_compute(r)
    start_pair_gathers(0, 0)
    start_pair_gathers(1, 2)

    @pl.loop(0, (NSTEP - 1) // 3)
    def main(j3):
        for u in range(3):
            step(3 * j3 + u, u)

    last = NSTEP - 1
    for m in range(2):
        gather_a(m, 2 * last + m).wait()
        gather_b(m, 2 * last + m).wait()
        scatter_d(m, 2 * last + m).start()
    for m in range(2):
        scatter_d(4 + m, 2 * last - 2 + m).wait()
        scatter_d(m, 2 * last + m).wait()


@functools.partial(
    pl.kernel,
    out_type=jax.ShapeDtypeStruct((BATCH, SEQ_LEN, 128), jnp.float32),
    mesh=plsc.VectorSubcoreMesh(core_axis_name="c", subcore_axis_name="s"),
    scratch_types=[
        pltpu.VMEM((TOK_PER_W,), jnp.int32),
        pltpu.VMEM((TOK_PER_W,), jnp.int32),
        pltpu.VMEM((NBUF, SEQ_LEN, EMBED_DIM), jnp.float32),
        pltpu.SemaphoreType.DMA((NBUF,)),
        pltpu.SemaphoreType.DMA((NBUF,)),
    ],
    compiler_params=pltpu.CompilerParams(
        use_tc_tiling_on_sc=False, needs_layout_passes=False
    ),
)
def _embed_lookup(inp_hbm, rep_hbm, out_hbm, inp_v, idx_v, rows_v, gsem, ssem):
    _sc_body(inp_hbm, rep_hbm, out_hbm, inp_v, idx_v, rows_v, gsem, ssem)


def kernel(input, weights):
    inp = input.astype(jnp.int32).reshape(BATCH * SEQ_LEN)
    rep = jnp.tile(weights[:TBL_ROWS].astype(jnp.float32), (NW, 1))
    return _embed_lookup(inp, rep)[:, :, :EMBED_DIM]
